# Initial kernel scaffold; baseline (speedup 1.0000x reference)
import functools

import jax
import jax.numpy as jnp
from jax import lax
from jax.experimental import pallas as pl
from jax.experimental.pallas import tpu as pltpu
from jax.experimental.pallas import tpu_sc as plsc

D = 64
CHUNK = 128
NC = 2
NS = 16
NW = NC * NS
B0, B1 = 4096, 200
N = B0 * B1
PER_W = N // NW
NCHUNK = PER_W // CHUNK
NBUF = 4
OUTER = NCHUNK // NBUF

_mesh = plsc.VectorSubcoreMesh(core_axis_name="c", subcore_axis_name="s")


@functools.partial(
    pl.kernel,
    mesh=_mesh,
    out_type=jax.ShapeDtypeStruct((N, D), jnp.float32),
    scratch_types=[
        pltpu.VMEM((NCHUNK, CHUNK), jnp.int32),
        pltpu.VMEM((NBUF * CHUNK, D), jnp.float32),
        pltpu.SemaphoreType.DMA,
    ],
)
def _pe_gather(x_hbm, pe_hbm, out_hbm, idx_v, rows_v, gsem):
    wid = lax.axis_index("s") * NC + lax.axis_index("c")
    base = wid * PER_W
    pltpu.sync_copy(x_hbm.at[wid], idx_v)

    def outer(G, carry):
        g0 = G * NBUF
        for b in range(NBUF):
            pltpu.make_async_copy(
                pe_hbm.at[idx_v.at[g0 + b]],
                rows_v.at[pl.ds(b * CHUNK, CHUNK)],
                gsem,
            ).start()
        for b in range(NBUF):
            pltpu.make_async_copy(
                pe_hbm.at[idx_v.at[g0 + b]],
                rows_v.at[pl.ds(b * CHUNK, CHUNK)],
                gsem,
            ).wait()
        pltpu.sync_copy(
            rows_v,
            out_hbm.at[pl.ds(base + g0 * CHUNK, NBUF * CHUNK)],
        )
        return carry

    lax.fori_loop(0, OUTER, outer, 0)


def kernel(x, pe):
    xi = x.reshape(NW, NCHUNK, CHUNK).astype(jnp.int32)
    out = _pe_gather(xi, pe)
    return out.reshape(B0, B1, D)


# speedup vs baseline: 4.8045x; 4.8045x over previous
import functools

import jax
import jax.numpy as jnp
from jax import lax
from jax.experimental import pallas as pl
from jax.experimental.pallas import tpu as pltpu
from jax.experimental.pallas import tpu_sc as plsc

D = 64
CHUNK = 128
NC = 2
NS = 16
NW = NC * NS
B0, B1 = 4096, 200
N = B0 * B1
PER_W = N // NW
NCHUNK = PER_W // CHUNK
NBUF = 4
OUTER = NCHUNK // NBUF

_mesh = plsc.VectorSubcoreMesh(core_axis_name="c", subcore_axis_name="s")


@functools.partial(
    pl.kernel,
    mesh=_mesh,
    out_type=jax.ShapeDtypeStruct((N, D), jnp.float32),
    scratch_types=[
        pltpu.VMEM((NCHUNK, CHUNK), jnp.int32),
        pltpu.VMEM((NBUF * CHUNK, D), jnp.float32),
        pltpu.SemaphoreType.DMA,
    ],
    compiler_params=pltpu.CompilerParams(use_tc_tiling_on_sc=False),
)
def _pe_gather(x_hbm, pe_hbm, out_hbm, idx_v, rows_v, gsem):
    wid = lax.axis_index("s") * NC + lax.axis_index("c")
    base = wid * PER_W
    pltpu.sync_copy(x_hbm.at[wid], idx_v)

    def outer(G, carry):
        g0 = G * NBUF
        for b in range(NBUF):
            pltpu.make_async_copy(
                pe_hbm.at[idx_v.at[g0 + b]],
                rows_v.at[pl.ds(b * CHUNK, CHUNK)],
                gsem,
            ).start()
        for b in range(NBUF):
            pltpu.make_async_copy(
                pe_hbm.at[idx_v.at[g0 + b]],
                rows_v.at[pl.ds(b * CHUNK, CHUNK)],
                gsem,
            ).wait()
        pltpu.sync_copy(
            rows_v,
            out_hbm.at[pl.ds(base + g0 * CHUNK, NBUF * CHUNK)],
        )
        return carry

    lax.fori_loop(0, OUTER, outer, 0)


def kernel(x, pe):
    xi = x.reshape(NW, NCHUNK, CHUNK).astype(jnp.int32)
    out = _pe_gather(xi, pe)
    return out.reshape(B0, B1, D)


# speedup vs baseline: 4.9613x; 1.0326x over previous
import functools

import jax
import jax.numpy as jnp
from jax import lax
from jax.experimental import pallas as pl
from jax.experimental.pallas import tpu as pltpu
from jax.experimental.pallas import tpu_sc as plsc

D = 64
CHUNK = 128
NC = 2
NS = 16
NW = NC * NS
B0, B1 = 4096, 200
N = B0 * B1
PER_W = N // NW
NCHUNK = PER_W // CHUNK
NBUF = 5
GROUP_ROWS = NBUF * CHUNK
NGROUPS = NCHUNK // NBUF

_mesh = plsc.VectorSubcoreMesh(core_axis_name="c", subcore_axis_name="s")


@functools.partial(
    pl.kernel,
    mesh=_mesh,
    out_type=jax.ShapeDtypeStruct((N, D), jnp.float32),
    scratch_types=[
        pltpu.VMEM((NCHUNK, CHUNK), jnp.int32),
        pltpu.VMEM((GROUP_ROWS, D), jnp.float32),
        pltpu.VMEM((GROUP_ROWS, D), jnp.float32),
        pltpu.SemaphoreType.DMA,
        pltpu.SemaphoreType.DMA,
    ],
    compiler_params=pltpu.CompilerParams(use_tc_tiling_on_sc=False),
)
def _pe_gather(x_hbm, pe_hbm, out_hbm, idx_v, rows0, rows1, gsem, wsem):
    wid = lax.axis_index("s") * NC + lax.axis_index("c")
    base = wid * PER_W
    pltpu.sync_copy(x_hbm.at[wid], idx_v)

    def g_copy(K, buf, b):
        return pltpu.make_async_copy(
            pe_hbm.at[idx_v.at[K * NBUF + b]],
            buf.at[pl.ds(b * CHUNK, CHUNK)],
            gsem,
        )

    def w_copy(K, buf):
        return pltpu.make_async_copy(
            buf,
            out_hbm.at[pl.ds(base + K * GROUP_ROWS, GROUP_ROWS)],
            wsem,
        )

    def start_g(K, buf):
        for b in range(NBUF):
            g_copy(K, buf, b).start()

    def drain_g(K, buf):
        for b in range(NBUF):
            g_copy(K, buf, b).wait()

    start_g(0, rows0)
    drain_g(0, rows0)
    start_g(1, rows1)
    w_copy(0, rows0).start()

    def body(KK, carry):
        k1 = 2 * KK + 1
        drain_g(k1, rows1)
        w_copy(k1 - 1, rows0).wait()
        start_g(k1 + 1, rows0)
        w_copy(k1, rows1).start()
        k2 = k1 + 1
        drain_g(k2, rows0)
        w_copy(k2 - 1, rows1).wait()
        start_g(k2 + 1, rows1)
        w_copy(k2, rows0).start()
        return carry

    lax.fori_loop(0, (NGROUPS - 2) // 2, body, 0)

    k_last = NGROUPS - 1
    drain_g(k_last, rows1)
    w_copy(k_last - 1, rows0).wait()
    w_copy(k_last, rows1).start()
    w_copy(k_last, rows1).wait()


def kernel(x, pe):
    xi = x.reshape(NW, NCHUNK, CHUNK).astype(jnp.int32)
    out = _pe_gather(xi, pe)
    return out.reshape(B0, B1, D)


# speedup vs baseline: 5.6038x; 1.1295x over previous
import functools

import jax
import jax.numpy as jnp
from jax import lax
from jax.experimental import pallas as pl
from jax.experimental.pallas import tpu as pltpu
from jax.experimental.pallas import tpu_sc as plsc

D = 64
CHUNK = 128
NC = 2
NS = 16
NW = NC * NS
B0, B1 = 4096, 200
N = B0 * B1
PER_W = N // NW
NCHUNK = PER_W // CHUNK
NBUF = 4
GROUP_ROWS = NBUF * CHUNK
NGROUPS = NCHUNK // NBUF

_mesh = plsc.VectorSubcoreMesh(core_axis_name="c", subcore_axis_name="s")


@functools.partial(
    pl.kernel,
    mesh=_mesh,
    out_type=jax.ShapeDtypeStruct((N, D), jnp.float32),
    scratch_types=[
        pltpu.VMEM((NCHUNK, CHUNK), jnp.int32),
        pltpu.VMEM((GROUP_ROWS, D), jnp.float32),
        pltpu.VMEM((GROUP_ROWS, D), jnp.float32),
        pltpu.VMEM_SHARED((8192, D), jnp.float32),
        pltpu.SemaphoreType.DMA,
        pltpu.SemaphoreType.DMA,
    ],
    compiler_params=pltpu.CompilerParams(use_tc_tiling_on_sc=False),
)
def _pe_gather(x_hbm, pe_hbm, out_hbm, idx_v, rows0, rows1, table_sp,
               gsem, wsem):
    s = lax.axis_index("s")
    wid = s * NC + lax.axis_index("c")
    base = wid * PER_W
    pltpu.sync_copy(pe_hbm.at[pl.ds(s * 512, 512)],
                    table_sp.at[pl.ds(s * 512, 512)])
    pltpu.sync_copy(x_hbm.at[wid], idx_v)
    plsc.subcore_barrier()

    def g_copy(K, buf, b):
        return pltpu.make_async_copy(
            table_sp.at[idx_v.at[K * NBUF + b]],
            buf.at[pl.ds(b * CHUNK, CHUNK)],
            gsem,
        )

    def w_copy(K, buf):
        return pltpu.make_async_copy(
            buf,
            out_hbm.at[pl.ds(base + K * GROUP_ROWS, GROUP_ROWS)],
            wsem,
        )

    def start_g(K, buf):
        for b in range(NBUF):
            g_copy(K, buf, b).start()

    def drain_g(K, buf):
        for b in range(NBUF):
            g_copy(K, buf, b).wait()

    start_g(0, rows0)
    drain_g(0, rows0)
    start_g(1, rows1)
    w_copy(0, rows0).start()

    def body(KK, carry):
        k1 = 2 * KK + 1
        drain_g(k1, rows1)
        w_copy(k1 - 1, rows0).wait()
        start_g(k1 + 1, rows0)
        w_copy(k1, rows1).start()
        k2 = k1 + 1
        drain_g(k2, rows0)
        w_copy(k2 - 1, rows1).wait()
        start_g(k2 + 1, rows1)
        w_copy(k2, rows0).start()
        return carry

    lax.fori_loop(0, (NGROUPS - 2) // 2, body, 0)

    k_last = NGROUPS - 1
    drain_g(k_last, rows1)
    w_copy(k_last - 1, rows0).wait()
    w_copy(k_last, rows1).start()
    w_copy(k_last, rows1).wait()


def kernel(x, pe):
    xi = x.reshape(NW, NCHUNK, CHUNK).astype(jnp.int32)
    out = _pe_gather(xi, pe)
    return out.reshape(B0, B1, D)


# speedup vs baseline: 9.7648x; 1.7425x over previous
import functools

import jax
import jax.numpy as jnp
from jax import lax
from jax.experimental import pallas as pl
from jax.experimental.pallas import tpu as pltpu
from jax.experimental.pallas import tpu_sc as plsc

D = 64
CHUNK = 128
NC = 2
NS = 16
NW = NC * NS
B0, B1 = 4096, 200
N = B0 * B1
PER_W = N // NW
NCHUNK = PER_W // CHUNK
V = 8192
VS = V // NS

_mesh = plsc.VectorSubcoreMesh(core_axis_name="c", subcore_axis_name="s")


@functools.partial(
    pl.kernel,
    mesh=_mesh,
    out_type=jax.ShapeDtypeStruct((N, D), jnp.float32),
    scratch_types=[
        pltpu.VMEM((NCHUNK, CHUNK), jnp.int32),
        pltpu.VMEM((CHUNK, D), jnp.float32),
        pltpu.VMEM((CHUNK, D), jnp.float32),
        pltpu.VMEM_SHARED((V, D), jnp.float32),
        pltpu.SemaphoreType.DMA,
        pltpu.SemaphoreType.DMA,
    ],
)
def _pe_gather(x_hbm, pe_hbm, out_hbm, idx_v, rows0, rows1, table_sp,
               gsem, wsem):
    s = lax.axis_index("s")
    wid = s * NC + lax.axis_index("c")
    base = wid * PER_W
    pltpu.sync_copy(pe_hbm.at[pl.ds(s * VS, VS)],
                    table_sp.at[pl.ds(s * VS, VS)])
    pltpu.sync_copy(x_hbm.at[wid], idx_v)
    plsc.subcore_barrier()

    def g_copy(K, buf):
        return pltpu.make_async_copy(
            table_sp.at[idx_v.at[K]],
            buf,
            gsem,
        )

    def w_copy(K, buf):
        return pltpu.make_async_copy(
            buf,
            out_hbm.at[pl.ds(base + K * CHUNK, CHUNK)],
            wsem,
        )

    g_copy(0, rows0).start()
    g_copy(0, rows0).wait()
    g_copy(1, rows1).start()
    w_copy(0, rows0).start()

    def body(KK, carry):
        k1 = 2 * KK + 1
        g_copy(k1, rows1).wait()
        w_copy(k1 - 1, rows0).wait()
        g_copy(k1 + 1, rows0).start()
        w_copy(k1, rows1).start()
        k2 = k1 + 1
        g_copy(k2, rows0).wait()
        w_copy(k2 - 1, rows1).wait()
        g_copy(k2 + 1, rows1).start()
        w_copy(k2, rows0).start()
        return carry

    lax.fori_loop(0, (NCHUNK - 2) // 2, body, 0)

    k_last = NCHUNK - 1
    g_copy(k_last, rows1).wait()
    w_copy(k_last - 1, rows0).wait()
    w_copy(k_last, rows1).start()
    w_copy(k_last, rows1).wait()


def kernel(x, pe):
    xi = x.reshape(NW, NCHUNK, CHUNK).astype(jnp.int32)
    out = _pe_gather(xi, pe)
    return out.reshape(B0, B1, D)
